# baseline (speedup 1.0000x reference)
import functools

import jax
import jax.numpy as jnp
from jax import lax
from jax.experimental import pallas as pl
from jax.experimental.pallas import tpu as pltpu
from jax.experimental.pallas import tpu_sc as plsc

_B, _C, _H, _W = 8, 256, 64, 64
_HW = _H * _W
_P = 100
_PPAD = 128


def _heads_kernel(x_ref, w1_ref, b1_ref, w2_ref, b2_ref,
                  scores_ref, head2_ref):
    x = x_ref[0]
    lane = jax.lax.broadcasted_iota(jnp.int32, (1, _HW), 1)
    xcol = lane % _W
    xm = x * (xcol <= _W - 2).astype(jnp.float32)
    xp_ = x * (xcol >= 1).astype(jnp.float32)
    srcs = {-1: xm, 0: x, 1: xp_}

    acc = jnp.zeros((_C, _HW), jnp.float32)
    for k in range(9):
        dy, dx = k // 3 - 1, k % 3 - 1
        src = srcs[dx]
        off = dy * _W + dx
        if off > 0:
            xs = jnp.concatenate(
                [src[:, off:], jnp.zeros((_C, off), jnp.float32)], axis=1)
        elif off < 0:
            xs = jnp.concatenate(
                [jnp.zeros((_C, -off), jnp.float32), src[:, :off]], axis=1)
        else:
            xs = src
        acc = acc + jax.lax.dot_general(
            w1_ref[k], xs, (((1,), (0,)), ((), ())),
            preferred_element_type=jnp.float32)
    hid = jnp.maximum(acc + b1_ref[:, 0:1], 0.0)

    head2 = jax.lax.dot_general(
        w2_ref[...], hid, (((1,), (0,)), ((), ())),
        preferred_element_type=jnp.float32) + b2_ref[:, 0:1]
    scores_ref[0] = jax.nn.sigmoid(head2[0:1, :])
    head2_ref[0] = head2


def _topk_decode_kernel(scores_ref, head2_ref,
                        vals_ref, eidx_ref, mask8_ref, bbox_ref, loc_ref,
                        idx_scr):
    b = pl.program_id(0)

    @pl.when(b == 0)
    def _topk():
        sc = scores_ref[:, 0, :]
        lane = jax.lax.broadcasted_iota(jnp.int32, (1, _HW), 1)
        lane_p = jax.lax.broadcasted_iota(jnp.int32, (1, _PPAD), 1)

        def body(t, carry):
            s, vals, idx = carry
            m = jnp.max(s, axis=1, keepdims=True)
            i = jnp.min(jnp.where(s == m, lane, jnp.int32(_HW)),
                        axis=1, keepdims=True)
            vals = jnp.where(lane_p == t, m, vals)
            idx = jnp.where(lane_p == t, i, idx)
            s = jnp.where(lane == i, -1.0, s)
            return s, vals, idx

        carry0 = (sc,
                  jnp.zeros((_B, _PPAD), jnp.float32),
                  jnp.full((_B, _PPAD), -1, jnp.int32))
        _, vals, idx = jax.lax.fori_loop(0, _P, body, carry0)
        vals_ref[...] = vals
        idx_scr[...] = idx

    idxr = idx_scr[pl.ds(b, 1), :]
    sub = jax.lax.broadcasted_iota(jnp.int32, (_HW, 1), 0)
    onehot_t = (sub == idxr).astype(jnp.float32)

    subc = jax.lax.broadcasted_iota(jnp.int32, (_C, 1), 0)
    eidx_ref[0] = ((jnp.maximum(idxr, 0) >> 3) + b * (_C * _HW // 8)
                   + subc * (_HW // 8))

    lane8 = jax.lax.broadcasted_iota(jnp.int32, (1, _PPAD * 8), 1)
    sub_p2 = jax.lax.broadcasted_iota(jnp.int32, (_PPAD, 1), 0)
    ups = ((lane8 >> 3) == sub_p2).astype(jnp.float32)
    idxmod_f = (idxr % 8).astype(jnp.float32)
    rmod = jax.lax.dot_general(
        idxmod_f, ups, (((1,), (0,)), ((), ())),
        preferred_element_type=jnp.float32)
    jmod_f = (lane8 % 8).astype(jnp.float32)
    mask8_ref[0] = (rmod == jmod_f).astype(jnp.float32)

    dsel = jax.lax.dot_general(
        head2_ref[0], onehot_t, (((1,), (0,)), ((), ())),
        preferred_element_type=jnp.float32)

    xx = idxr % _W
    yy = idxr // _W
    dxv, dyv, dwv, dhv = dsel[1:2], dsel[2:3], dsel[3:4], dsel[4:5]
    cx = xx.astype(jnp.float32) * 8.0 + dxv * 8.0
    cy = yy.astype(jnp.float32) * 8.0 + dyv * 8.0
    wv = jnp.exp(dwv) * 8.0
    hv = jnp.exp(dhv) * 8.0

    bbox_ref[0, 0:1, :] = cx - wv * 0.5
    bbox_ref[0, 1:2, :] = cy - hv * 0.5
    bbox_ref[0, 2:3, :] = cx + wv * 0.5
    bbox_ref[0, 3:4, :] = cy + hv * 0.5
    bbox_ref[0, 4:8, :] = jnp.zeros((4, _PPAD), jnp.float32)
    loc_ref[0, 0:1, :] = xx
    loc_ref[0, 1:2, :] = yy
    loc_ref[0, 2:8, :] = jnp.zeros((6, _PPAD), jnp.int32)


def _select_kernel(gath_ref, mask8_ref, featsT_ref):
    gm = gath_ref[0] * mask8_ref[0]
    sub8 = jax.lax.broadcasted_iota(jnp.int32, (_PPAD * 8, 1), 0)
    lane_p = jax.lax.broadcasted_iota(jnp.int32, (1, _PPAD), 1)
    fold = ((sub8 >> 3) == lane_p).astype(jnp.float32)
    featsT_ref[0] = jax.lax.dot_general(
        gm, fold, (((1,), (0,)), ((), ())),
        preferred_element_type=jnp.float32)


def _make_sc_gather():
    info = plsc.get_sparse_core_info()
    nc, ns = info.num_cores, info.num_subcores
    nw = nc * ns
    rows_total = _B * _PPAD * _C
    rows_per_w = rows_total // nw
    cw = 128
    chunks = rows_per_w // cw
    mesh = plsc.VectorSubcoreMesh(core_axis_name="c", subcore_axis_name="s")

    total_chunks = nw * chunks

    @functools.partial(
        pl.kernel, mesh=mesh,
        compiler_params=pltpu.CompilerParams(use_tc_tiling_on_sc=False),
        out_type=jax.ShapeDtypeStruct((total_chunks, cw, 8), jnp.float32),
        scratch_types=[
            pltpu.VMEM((chunks, cw), jnp.int32),
            pltpu.VMEM((chunks, cw, 8), jnp.float32),
            pltpu.SemaphoreType.DMA,
        ],
    )
    def sc_gather(table_hbm, eidx_hbm, out_hbm, eidx_v, rows_v, sem):
        wid = lax.axis_index("s") * nc + lax.axis_index("c")
        base = wid * chunks
        pltpu.sync_copy(eidx_hbm.at[pl.ds(base, chunks)], eidx_v)
        copies = []
        for i in range(chunks):
            copies.append(pltpu.async_copy(
                table_hbm.at[eidx_v.at[i]], rows_v.at[i], sem))
        for c in copies:
            c.wait()
        pltpu.sync_copy(rows_v, out_hbm.at[pl.ds(base, chunks)])

    return sc_gather


_sc_gather = _make_sc_gather()


def kernel(features, obj_w1, obj_b1, obj_w2, obj_b2,
           box_w1, box_b1, box_w2, box_b2):
    xf = features.reshape(_B, _C, _HW)

    w1c = jnp.concatenate([obj_w1, box_w1], axis=0)
    w1t = jnp.transpose(w1c, (2, 3, 0, 1)).reshape(9, _C, _C)
    b1c = jnp.concatenate([obj_b1, box_b1], axis=0).reshape(_C, 1)
    b1p = jnp.broadcast_to(b1c, (_C, 128))

    w2 = jnp.zeros((8, _C), jnp.float32)
    w2 = w2.at[0, :128].set(obj_w2.reshape(128))
    w2 = w2.at[1:5, 128:].set(box_w2.reshape(4, 128))
    b2 = jnp.zeros((8,), jnp.float32)
    b2 = b2.at[0].set(obj_b2[0]).at[1:5].set(box_b2)
    b2p = jnp.broadcast_to(b2.reshape(8, 1), (8, 128))

    scores3, head2o = pl.pallas_call(
        _heads_kernel,
        grid=(_B,),
        in_specs=[
            pl.BlockSpec((1, _C, _HW), lambda b: (b, 0, 0)),
            pl.BlockSpec((9, _C, _C), lambda b: (0, 0, 0)),
            pl.BlockSpec((_C, 128), lambda b: (0, 0)),
            pl.BlockSpec((8, _C), lambda b: (0, 0)),
            pl.BlockSpec((8, 128), lambda b: (0, 0)),
        ],
        out_specs=[
            pl.BlockSpec((1, 1, _HW), lambda b: (b, 0, 0)),
            pl.BlockSpec((1, 8, _HW), lambda b: (b, 0, 0)),
        ],
        out_shape=[
            jax.ShapeDtypeStruct((_B, 1, _HW), jnp.float32),
            jax.ShapeDtypeStruct((_B, 8, _HW), jnp.float32),
        ],
    )(xf, w1t, b1p, w2, b2p)

    vals, eidx, mask8, bbox_o, loc_o = pl.pallas_call(
        _topk_decode_kernel,
        grid=(_B,),
        in_specs=[
            pl.BlockSpec((_B, 1, _HW), lambda b: (0, 0, 0)),
            pl.BlockSpec((1, 8, _HW), lambda b: (b, 0, 0)),
        ],
        out_specs=[
            pl.BlockSpec((_B, _PPAD), lambda b: (0, 0)),
            pl.BlockSpec((1, _C, _PPAD), lambda b: (b, 0, 0)),
            pl.BlockSpec((1, 1, _PPAD * 8), lambda b: (b, 0, 0)),
            pl.BlockSpec((1, 8, _PPAD), lambda b: (b, 0, 0)),
            pl.BlockSpec((1, 8, _PPAD), lambda b: (b, 0, 0)),
        ],
        out_shape=[
            jax.ShapeDtypeStruct((_B, _PPAD), jnp.float32),
            jax.ShapeDtypeStruct((_B, _C, _PPAD), jnp.int32),
            jax.ShapeDtypeStruct((_B, 1, _PPAD * 8), jnp.float32),
            jax.ShapeDtypeStruct((_B, 8, _PPAD), jnp.float32),
            jax.ShapeDtypeStruct((_B, 8, _PPAD), jnp.int32),
        ],
        scratch_shapes=[pltpu.VMEM((_B, _PPAD), jnp.int32)],
    )(scores3, head2o)

    gath = _sc_gather(features.reshape(_B * _C * _HW // 8, 8),
                      eidx.reshape(_B * _C * _PPAD // 128, 128))
    gath = gath.reshape(_B, _C, _PPAD * 8)

    featsT_o = pl.pallas_call(
        _select_kernel,
        grid=(_B,),
        in_specs=[
            pl.BlockSpec((1, _C, _PPAD * 8), lambda b: (b, 0, 0)),
            pl.BlockSpec((1, 1, _PPAD * 8), lambda b: (b, 0, 0)),
        ],
        out_specs=pl.BlockSpec((1, _C, _PPAD), lambda b: (b, 0, 0)),
        out_shape=jax.ShapeDtypeStruct((_B, _C, _PPAD), jnp.float32),
    )(gath, mask8)
    feats = featsT_o[:, :, :_P].transpose(0, 2, 1)

    top_scores = vals[:, :_P]
    bboxes = bbox_o[:, :4, :_P].transpose(0, 2, 1)
    locations = loc_o[:, :2, :_P].transpose(0, 2, 1)
    return bboxes, top_scores, feats, locations


# speedup vs baseline: 1.6762x; 1.6762x over previous
import jax
import jax.numpy as jnp
from jax.experimental import pallas as pl
from jax.experimental.pallas import tpu as pltpu

_B, _C, _H, _W = 8, 256, 64, 64
_HW = _H * _W
_P = 100
_PPAD = 128


def _heads_kernel(x_ref, w1_ref, b1_ref, w2_ref, b2_ref,
                  scores_ref, head2_ref):
    x = x_ref[0]
    lane = jax.lax.broadcasted_iota(jnp.int32, (1, _HW), 1)
    xcol = lane % _W
    xm = x * (xcol <= _W - 2).astype(jnp.float32)
    xp_ = x * (xcol >= 1).astype(jnp.float32)
    srcs = {-1: xm, 0: x, 1: xp_}

    acc = jnp.zeros((_C, _HW), jnp.float32)
    for k in range(9):
        dy, dx = k // 3 - 1, k % 3 - 1
        src = srcs[dx]
        off = dy * _W + dx
        if off > 0:
            xs = jnp.concatenate(
                [src[:, off:], jnp.zeros((_C, off), jnp.float32)], axis=1)
        elif off < 0:
            xs = jnp.concatenate(
                [jnp.zeros((_C, -off), jnp.float32), src[:, :off]], axis=1)
        else:
            xs = src
        acc = acc + jax.lax.dot_general(
            w1_ref[k], xs, (((1,), (0,)), ((), ())),
            preferred_element_type=jnp.float32)
    hid = jnp.maximum(acc + b1_ref[:, 0:1], 0.0)

    head2 = jax.lax.dot_general(
        w2_ref[...], hid, (((1,), (0,)), ((), ())),
        preferred_element_type=jnp.float32) + b2_ref[:, 0:1]
    scores_ref[0] = jax.nn.sigmoid(head2[0:1, :])
    head2_ref[0] = head2


def _topk_gather_kernel(scores_ref, x_ref, head2_ref,
                        vals_ref, bbox_ref, featsT_ref, loc_ref,
                        idx_scr):
    b = pl.program_id(0)

    @pl.when(b == 0)
    def _topk():
        sc = scores_ref[:, 0, :]
        lane = jax.lax.broadcasted_iota(jnp.int32, (1, _HW), 1)
        lane_p = jax.lax.broadcasted_iota(jnp.int32, (1, _PPAD), 1)

        def body(t, carry):
            s, vals, idx = carry
            m = jnp.max(s, axis=1, keepdims=True)
            i = jnp.min(jnp.where(s == m, lane, jnp.int32(_HW)),
                        axis=1, keepdims=True)
            vals = jnp.where(lane_p == t, m, vals)
            idx = jnp.where(lane_p == t, i, idx)
            s = jnp.where(lane == i, -1.0, s)
            return s, vals, idx

        carry0 = (sc,
                  jnp.zeros((_B, _PPAD), jnp.float32),
                  jnp.full((_B, _PPAD), -1, jnp.int32))
        _, vals, idx = jax.lax.fori_loop(0, _P, body, carry0)
        vals_ref[...] = vals
        idx_scr[...] = idx

    idxr = idx_scr[pl.ds(b, 1), :]
    sub = jax.lax.broadcasted_iota(jnp.int32, (_HW, 1), 0)
    onehot_t = (sub == idxr).astype(jnp.float32)

    fsel_t = jax.lax.dot_general(
        x_ref[0], onehot_t, (((1,), (0,)), ((), ())),
        preferred_element_type=jnp.float32)
    dsel = jax.lax.dot_general(
        head2_ref[0], onehot_t, (((1,), (0,)), ((), ())),
        preferred_element_type=jnp.float32)

    xx = idxr % _W
    yy = idxr // _W
    dxv, dyv, dwv, dhv = dsel[1:2], dsel[2:3], dsel[3:4], dsel[4:5]
    cx = xx.astype(jnp.float32) * 8.0 + dxv * 8.0
    cy = yy.astype(jnp.float32) * 8.0 + dyv * 8.0
    wv = jnp.exp(dwv) * 8.0
    hv = jnp.exp(dhv) * 8.0

    bbox_ref[0, 0:1, :] = cx - wv * 0.5
    bbox_ref[0, 1:2, :] = cy - hv * 0.5
    bbox_ref[0, 2:3, :] = cx + wv * 0.5
    bbox_ref[0, 3:4, :] = cy + hv * 0.5
    bbox_ref[0, 4:8, :] = jnp.zeros((4, _PPAD), jnp.float32)
    featsT_ref[0] = fsel_t
    loc_ref[0, 0:1, :] = xx
    loc_ref[0, 1:2, :] = yy
    loc_ref[0, 2:8, :] = jnp.zeros((6, _PPAD), jnp.int32)


def kernel(features, obj_w1, obj_b1, obj_w2, obj_b2,
           box_w1, box_b1, box_w2, box_b2):
    xf = features.reshape(_B, _C, _HW)

    w1c = jnp.concatenate([obj_w1, box_w1], axis=0)
    w1t = jnp.transpose(w1c, (2, 3, 0, 1)).reshape(9, _C, _C)
    b1c = jnp.concatenate([obj_b1, box_b1], axis=0).reshape(_C, 1)
    b1p = jnp.broadcast_to(b1c, (_C, 128))

    w2 = jnp.zeros((8, _C), jnp.float32)
    w2 = w2.at[0, :128].set(obj_w2.reshape(128))
    w2 = w2.at[1:5, 128:].set(box_w2.reshape(4, 128))
    b2 = jnp.zeros((8,), jnp.float32)
    b2 = b2.at[0].set(obj_b2[0]).at[1:5].set(box_b2)
    b2p = jnp.broadcast_to(b2.reshape(8, 1), (8, 128))

    scores3, head2o = pl.pallas_call(
        _heads_kernel,
        grid=(_B,),
        in_specs=[
            pl.BlockSpec((1, _C, _HW), lambda b: (b, 0, 0)),
            pl.BlockSpec((9, _C, _C), lambda b: (0, 0, 0)),
            pl.BlockSpec((_C, 128), lambda b: (0, 0)),
            pl.BlockSpec((8, _C), lambda b: (0, 0)),
            pl.BlockSpec((8, 128), lambda b: (0, 0)),
        ],
        out_specs=[
            pl.BlockSpec((1, 1, _HW), lambda b: (b, 0, 0)),
            pl.BlockSpec((1, 8, _HW), lambda b: (b, 0, 0)),
        ],
        out_shape=[
            jax.ShapeDtypeStruct((_B, 1, _HW), jnp.float32),
            jax.ShapeDtypeStruct((_B, 8, _HW), jnp.float32),
        ],
    )(xf, w1t, b1p, w2, b2p)

    vals, bbox_o, featsT_o, loc_o = pl.pallas_call(
        _topk_gather_kernel,
        grid=(_B,),
        in_specs=[
            pl.BlockSpec((_B, 1, _HW), lambda b: (0, 0, 0)),
            pl.BlockSpec((1, _C, _HW), lambda b: (b, 0, 0)),
            pl.BlockSpec((1, 8, _HW), lambda b: (b, 0, 0)),
        ],
        out_specs=[
            pl.BlockSpec((_B, _PPAD), lambda b: (0, 0)),
            pl.BlockSpec((1, 8, _PPAD), lambda b: (b, 0, 0)),
            pl.BlockSpec((1, _C, _PPAD), lambda b: (b, 0, 0)),
            pl.BlockSpec((1, 8, _PPAD), lambda b: (b, 0, 0)),
        ],
        out_shape=[
            jax.ShapeDtypeStruct((_B, _PPAD), jnp.float32),
            jax.ShapeDtypeStruct((_B, 8, _PPAD), jnp.float32),
            jax.ShapeDtypeStruct((_B, _C, _PPAD), jnp.float32),
            jax.ShapeDtypeStruct((_B, 8, _PPAD), jnp.int32),
        ],
        scratch_shapes=[pltpu.VMEM((_B, _PPAD), jnp.int32)],
    )(scores3, xf, head2o)

    top_scores = vals[:, :_P]
    bboxes = bbox_o[:, :4, :_P].transpose(0, 2, 1)
    feats = featsT_o[:, :, :_P].transpose(0, 2, 1)
    locations = loc_o[:, :2, :_P].transpose(0, 2, 1)
    return bboxes, top_scores, feats, locations


# speedup vs baseline: 1.7687x; 1.0551x over previous
import jax
import jax.numpy as jnp
from jax.experimental import pallas as pl
from jax.experimental.pallas import tpu as pltpu

_B, _C, _H, _W = 8, 256, 64, 64
_HW = _H * _W
_P = 100
_PPAD = 128


def _heads_kernel(x_ref, w1_ref, b1_ref, w2_ref, b2_ref,
                  scores_ref, head2_ref):
    x = x_ref[0]
    lane = jax.lax.broadcasted_iota(jnp.int32, (1, _HW), 1)
    xcol = lane % _W
    xm = x * (xcol <= _W - 2).astype(jnp.float32)
    xp_ = x * (xcol >= 1).astype(jnp.float32)
    srcs = {-1: xm, 0: x, 1: xp_}

    acc = jnp.zeros((_C, _HW), jnp.float32)
    for k in range(9):
        dy, dx = k // 3 - 1, k % 3 - 1
        src = srcs[dx]
        off = dy * _W + dx
        if off > 0:
            xs = jnp.concatenate(
                [src[:, off:], jnp.zeros((_C, off), jnp.float32)], axis=1)
        elif off < 0:
            xs = jnp.concatenate(
                [jnp.zeros((_C, -off), jnp.float32), src[:, :off]], axis=1)
        else:
            xs = src
        acc = acc + jax.lax.dot_general(
            w1_ref[k], xs, (((1,), (0,)), ((), ())),
            preferred_element_type=jnp.float32)
    hid = jnp.maximum(acc + b1_ref[:, 0:1], 0.0)

    head2 = jax.lax.dot_general(
        w2_ref[...], hid, (((1,), (0,)), ((), ())),
        preferred_element_type=jnp.float32) + b2_ref[:, 0:1]
    scores_ref[0] = jax.nn.sigmoid(head2[0:1, :])
    head2_ref[0] = head2


def _topk_gather_kernel(scores_ref, x_ref, head2_ref,
                        vals_ref, bbox_ref, featsT_ref, loc_ref,
                        idx_scr):
    b = pl.program_id(0)

    @pl.when(b == 0)
    def _topk():
        sc = scores_ref[:, 0, :]
        lane = jax.lax.broadcasted_iota(jnp.int32, (1, _HW), 1)
        lane_p = jax.lax.broadcasted_iota(jnp.int32, (1, _PPAD), 1)

        def body(t, carry):
            s, vals, idx = carry
            m = jnp.max(s, axis=1, keepdims=True)
            i = jnp.argmax(s, axis=1, keepdims=True).astype(jnp.int32)
            vals = jnp.where(lane_p == t, m, vals)
            idx = jnp.where(lane_p == t, i, idx)
            s = jnp.where(lane == i, -1.0, s)
            return s, vals, idx

        carry0 = (sc,
                  jnp.zeros((_B, _PPAD), jnp.float32),
                  jnp.full((_B, _PPAD), -1, jnp.int32))
        _, vals, idx = jax.lax.fori_loop(0, _P, body, carry0)
        vals_ref[...] = vals
        idx_scr[...] = idx

    idxr = idx_scr[pl.ds(b, 1), :]
    sub = jax.lax.broadcasted_iota(jnp.int32, (_HW, 1), 0)
    onehot_t = (sub == idxr).astype(jnp.float32)

    fsel_t = jax.lax.dot_general(
        x_ref[0], onehot_t, (((1,), (0,)), ((), ())),
        preferred_element_type=jnp.float32)
    dsel = jax.lax.dot_general(
        head2_ref[0], onehot_t, (((1,), (0,)), ((), ())),
        preferred_element_type=jnp.float32)

    xx = idxr % _W
    yy = idxr // _W
    dxv, dyv, dwv, dhv = dsel[1:2], dsel[2:3], dsel[3:4], dsel[4:5]
    cx = xx.astype(jnp.float32) * 8.0 + dxv * 8.0
    cy = yy.astype(jnp.float32) * 8.0 + dyv * 8.0
    wv = jnp.exp(dwv) * 8.0
    hv = jnp.exp(dhv) * 8.0

    bbox_ref[0, 0:1, :] = cx - wv * 0.5
    bbox_ref[0, 1:2, :] = cy - hv * 0.5
    bbox_ref[0, 2:3, :] = cx + wv * 0.5
    bbox_ref[0, 3:4, :] = cy + hv * 0.5
    bbox_ref[0, 4:8, :] = jnp.zeros((4, _PPAD), jnp.float32)
    featsT_ref[0] = fsel_t
    loc_ref[0, 0:1, :] = xx
    loc_ref[0, 1:2, :] = yy
    loc_ref[0, 2:8, :] = jnp.zeros((6, _PPAD), jnp.int32)


def kernel(features, obj_w1, obj_b1, obj_w2, obj_b2,
           box_w1, box_b1, box_w2, box_b2):
    xf = features.reshape(_B, _C, _HW)

    w1c = jnp.concatenate([obj_w1, box_w1], axis=0)
    w1t = jnp.transpose(w1c, (2, 3, 0, 1)).reshape(9, _C, _C)
    b1c = jnp.concatenate([obj_b1, box_b1], axis=0).reshape(_C, 1)
    b1p = jnp.broadcast_to(b1c, (_C, 128))

    w2 = jnp.zeros((8, _C), jnp.float32)
    w2 = w2.at[0, :128].set(obj_w2.reshape(128))
    w2 = w2.at[1:5, 128:].set(box_w2.reshape(4, 128))
    b2 = jnp.zeros((8,), jnp.float32)
    b2 = b2.at[0].set(obj_b2[0]).at[1:5].set(box_b2)
    b2p = jnp.broadcast_to(b2.reshape(8, 1), (8, 128))

    scores3, head2o = pl.pallas_call(
        _heads_kernel,
        grid=(_B,),
        in_specs=[
            pl.BlockSpec((1, _C, _HW), lambda b: (b, 0, 0)),
            pl.BlockSpec((9, _C, _C), lambda b: (0, 0, 0)),
            pl.BlockSpec((_C, 128), lambda b: (0, 0)),
            pl.BlockSpec((8, _C), lambda b: (0, 0)),
            pl.BlockSpec((8, 128), lambda b: (0, 0)),
        ],
        out_specs=[
            pl.BlockSpec((1, 1, _HW), lambda b: (b, 0, 0)),
            pl.BlockSpec((1, 8, _HW), lambda b: (b, 0, 0)),
        ],
        out_shape=[
            jax.ShapeDtypeStruct((_B, 1, _HW), jnp.float32),
            jax.ShapeDtypeStruct((_B, 8, _HW), jnp.float32),
        ],
    )(xf, w1t, b1p, w2, b2p)

    vals, bbox_o, featsT_o, loc_o = pl.pallas_call(
        _topk_gather_kernel,
        grid=(_B,),
        in_specs=[
            pl.BlockSpec((_B, 1, _HW), lambda b: (0, 0, 0)),
            pl.BlockSpec((1, _C, _HW), lambda b: (b, 0, 0)),
            pl.BlockSpec((1, 8, _HW), lambda b: (b, 0, 0)),
        ],
        out_specs=[
            pl.BlockSpec((_B, _PPAD), lambda b: (0, 0)),
            pl.BlockSpec((1, 8, _PPAD), lambda b: (b, 0, 0)),
            pl.BlockSpec((1, _C, _PPAD), lambda b: (b, 0, 0)),
            pl.BlockSpec((1, 8, _PPAD), lambda b: (b, 0, 0)),
        ],
        out_shape=[
            jax.ShapeDtypeStruct((_B, _PPAD), jnp.float32),
            jax.ShapeDtypeStruct((_B, 8, _PPAD), jnp.float32),
            jax.ShapeDtypeStruct((_B, _C, _PPAD), jnp.float32),
            jax.ShapeDtypeStruct((_B, 8, _PPAD), jnp.int32),
        ],
        scratch_shapes=[pltpu.VMEM((_B, _PPAD), jnp.int32)],
    )(scores3, xf, head2o)

    top_scores = vals[:, :_P]
    bboxes = bbox_o[:, :4, :_P].transpose(0, 2, 1)
    feats = featsT_o[:, :, :_P].transpose(0, 2, 1)
    locations = loc_o[:, :2, :_P].transpose(0, 2, 1)
    return bboxes, top_scores, feats, locations


# speedup vs baseline: 1.7699x; 1.0007x over previous
import jax
import jax.numpy as jnp
from jax.experimental import pallas as pl
from jax.experimental.pallas import tpu as pltpu

_B, _C, _H, _W = 8, 256, 64, 64
_HW = _H * _W
_P = 100
_PPAD = 128


def _heads_kernel(x_ref, w1_ref, b1_ref, w2_ref, b2_ref,
                  scores_ref, head2_ref):
    x = x_ref[0]
    lane = jax.lax.broadcasted_iota(jnp.int32, (1, _HW), 1)
    xcol = lane % _W
    xm = x * (xcol <= _W - 2).astype(jnp.float32)
    xp_ = x * (xcol >= 1).astype(jnp.float32)
    srcs = {-1: xm, 0: x, 1: xp_}

    acc = jnp.zeros((_C, _HW), jnp.float32)
    for k in range(9):
        dy, dx = k // 3 - 1, k % 3 - 1
        src = srcs[dx]
        off = dy * _W + dx
        if off > 0:
            xs = jnp.concatenate(
                [src[:, off:], jnp.zeros((_C, off), jnp.float32)], axis=1)
        elif off < 0:
            xs = jnp.concatenate(
                [jnp.zeros((_C, -off), jnp.float32), src[:, :off]], axis=1)
        else:
            xs = src
        acc = acc + jax.lax.dot_general(
            w1_ref[k], xs, (((1,), (0,)), ((), ())),
            preferred_element_type=jnp.float32)
    hid = jnp.maximum(acc + b1_ref[:, 0:1], 0.0)

    head2 = jax.lax.dot_general(
        w2_ref[...], hid, (((1,), (0,)), ((), ())),
        preferred_element_type=jnp.float32) + b2_ref[:, 0:1]
    scores_ref[0] = jax.nn.sigmoid(head2[0:1, :])
    head2_ref[0] = head2


def _topk_gather_kernel(scores_ref, x_ref, head2_ref,
                        vals_ref, bbox_ref, featsT_ref, loc_ref,
                        idx_scr):
    b = pl.program_id(0)

    @pl.when(b == 0)
    def _topk():
        sc = scores_ref[:, 0, :]
        lane = jax.lax.broadcasted_iota(jnp.int32, (1, _HW), 1)
        lane_p = jax.lax.broadcasted_iota(jnp.int32, (1, _PPAD), 1)

        def body(t, carry):
            s, idx = carry
            i = jnp.argmax(s, axis=1, keepdims=True).astype(jnp.int32)
            idx = jnp.where(lane_p == t, i, idx)
            s = jnp.where(lane == i, -1.0, s)
            return s, idx

        carry0 = (sc, jnp.full((_B, _PPAD), -1, jnp.int32))
        _, idx = jax.lax.fori_loop(0, _P, body, carry0)
        idx_scr[...] = idx

    idxr = idx_scr[pl.ds(b, 1), :]
    sub = jax.lax.broadcasted_iota(jnp.int32, (_HW, 1), 0)
    onehot_t = (sub == idxr).astype(jnp.float32)

    fsel_t = jax.lax.dot_general(
        x_ref[0], onehot_t, (((1,), (0,)), ((), ())),
        preferred_element_type=jnp.float32)
    dsel = jax.lax.dot_general(
        head2_ref[0], onehot_t, (((1,), (0,)), ((), ())),
        preferred_element_type=jnp.float32)

    vals_ref[0, 0:1, :] = jax.nn.sigmoid(dsel[0:1, :])

    xx = idxr % _W
    yy = idxr // _W
    dxv, dyv, dwv, dhv = dsel[1:2], dsel[2:3], dsel[3:4], dsel[4:5]
    cx = xx.astype(jnp.float32) * 8.0 + dxv * 8.0
    cy = yy.astype(jnp.float32) * 8.0 + dyv * 8.0
    wv = jnp.exp(dwv) * 8.0
    hv = jnp.exp(dhv) * 8.0

    bbox_ref[0, 0:1, :] = cx - wv * 0.5
    bbox_ref[0, 1:2, :] = cy - hv * 0.5
    bbox_ref[0, 2:3, :] = cx + wv * 0.5
    bbox_ref[0, 3:4, :] = cy + hv * 0.5
    bbox_ref[0, 4:8, :] = jnp.zeros((4, _PPAD), jnp.float32)
    featsT_ref[0] = fsel_t
    loc_ref[0, 0:1, :] = xx
    loc_ref[0, 1:2, :] = yy
    loc_ref[0, 2:8, :] = jnp.zeros((6, _PPAD), jnp.int32)


def kernel(features, obj_w1, obj_b1, obj_w2, obj_b2,
           box_w1, box_b1, box_w2, box_b2):
    xf = features.reshape(_B, _C, _HW)

    w1c = jnp.concatenate([obj_w1, box_w1], axis=0)
    w1t = jnp.transpose(w1c, (2, 3, 0, 1)).reshape(9, _C, _C)
    b1c = jnp.concatenate([obj_b1, box_b1], axis=0).reshape(_C, 1)
    b1p = jnp.broadcast_to(b1c, (_C, 128))

    w2 = jnp.zeros((8, _C), jnp.float32)
    w2 = w2.at[0, :128].set(obj_w2.reshape(128))
    w2 = w2.at[1:5, 128:].set(box_w2.reshape(4, 128))
    b2 = jnp.zeros((8,), jnp.float32)
    b2 = b2.at[0].set(obj_b2[0]).at[1:5].set(box_b2)
    b2p = jnp.broadcast_to(b2.reshape(8, 1), (8, 128))

    scores3, head2o = pl.pallas_call(
        _heads_kernel,
        grid=(_B,),
        in_specs=[
            pl.BlockSpec((1, _C, _HW), lambda b: (b, 0, 0)),
            pl.BlockSpec((9, _C, _C), lambda b: (0, 0, 0)),
            pl.BlockSpec((_C, 128), lambda b: (0, 0)),
            pl.BlockSpec((8, _C), lambda b: (0, 0)),
            pl.BlockSpec((8, 128), lambda b: (0, 0)),
        ],
        out_specs=[
            pl.BlockSpec((1, 1, _HW), lambda b: (b, 0, 0)),
            pl.BlockSpec((1, 8, _HW), lambda b: (b, 0, 0)),
        ],
        out_shape=[
            jax.ShapeDtypeStruct((_B, 1, _HW), jnp.float32),
            jax.ShapeDtypeStruct((_B, 8, _HW), jnp.float32),
        ],
    )(xf, w1t, b1p, w2, b2p)

    vals, bbox_o, featsT_o, loc_o = pl.pallas_call(
        _topk_gather_kernel,
        grid=(_B,),
        in_specs=[
            pl.BlockSpec((_B, 1, _HW), lambda b: (0, 0, 0)),
            pl.BlockSpec((1, _C, _HW), lambda b: (b, 0, 0)),
            pl.BlockSpec((1, 8, _HW), lambda b: (b, 0, 0)),
        ],
        out_specs=[
            pl.BlockSpec((1, 1, _PPAD), lambda b: (b, 0, 0)),
            pl.BlockSpec((1, 8, _PPAD), lambda b: (b, 0, 0)),
            pl.BlockSpec((1, _C, _PPAD), lambda b: (b, 0, 0)),
            pl.BlockSpec((1, 8, _PPAD), lambda b: (b, 0, 0)),
        ],
        out_shape=[
            jax.ShapeDtypeStruct((_B, 1, _PPAD), jnp.float32),
            jax.ShapeDtypeStruct((_B, 8, _PPAD), jnp.float32),
            jax.ShapeDtypeStruct((_B, _C, _PPAD), jnp.float32),
            jax.ShapeDtypeStruct((_B, 8, _PPAD), jnp.int32),
        ],
        scratch_shapes=[pltpu.VMEM((_B, _PPAD), jnp.int32)],
    )(scores3, xf, head2o)

    top_scores = vals[:, 0, :_P]
    bboxes = bbox_o[:, :4, :_P].transpose(0, 2, 1)
    feats = featsT_o[:, :, :_P].transpose(0, 2, 1)
    locations = loc_o[:, :2, :_P].transpose(0, 2, 1)
    return bboxes, top_scores, feats, locations
